# baseline (speedup 1.0000x reference)
import dataclasses
import functools

import jax
import jax.numpy as jnp
from jax import lax
from jax.experimental import pallas as pl
from jax.experimental.pallas import tpu as pltpu
from jax.experimental.pallas import tpu_sc as plsc

N = 10000
D = 128
OUT = 64
E = 320000
NC, NS = 2, 16
NW = NC * NS
CHUNK = 128
E_PAD = 327680
NCHUNK = E_PAD // CHUNK
CPW_H = NCHUNK // NW
CPS = NCHUNK // NS
EPS = CPS * CHUNK
NBUF = 4
N_PAD = 10240
HALF = N_PAD // NC
ACC_ROWS = HALF + CHUNK
STRIPE = HALF // NS
NPM = NBUF * CHUNK
LCAP = EPS + NPM
SG = 4
CPG = CPS // SG
SHIFT = 14
DMASK = (1 << SHIFT) - 1
RB = 2048
GRID = N_PAD // RB

_mesh = plsc.VectorSubcoreMesh(core_axis_name="c", subcore_axis_name="s")

_sc_cp = pltpu.CompilerParams()
if "needs_layout_passes" in pltpu.CompilerParams.__dataclass_fields__:
    _sc_cp = dataclasses.replace(_sc_cp, needs_layout_passes=False)
_sc_cp_mv = dataclasses.replace(_sc_cp, use_tc_tiling_on_sc=False)


@functools.partial(
    pl.kernel,
    mesh=_mesh,
    out_type=(
        jax.ShapeDtypeStruct((NW, N_PAD), jnp.float32),
        jax.ShapeDtypeStruct((NW, LCAP), jnp.int32),
        jax.ShapeDtypeStruct((NW, 16), jnp.int32),
    ),
    compiler_params=_sc_cp,
    scratch_types=[
        pltpu.VMEM((CPG, CHUNK), jnp.int32),
        pltpu.VMEM((CPG, CHUNK), jnp.int32),
        pltpu.VMEM((LCAP,), jnp.int32),
        pltpu.VMEM((N_PAD + 16,), jnp.float32),
        pltpu.VMEM((16,), jnp.int32),
        pltpu.SMEM((1,), jnp.int32),
    ],
)
def _sc_prep(src_hbm, dst_hbm, hist_out, list_out, cnt_out,
             sbuf, dbuf, plist, hist_v, cnt_v, cur):
    cid = lax.axis_index("c")
    sid = lax.axis_index("s")
    wid = cid * NS + sid
    base_row = cid * HALF
    iota16 = lax.iota(jnp.int32, 16)
    zeros16 = jnp.zeros((16,), jnp.float32)
    ones16 = jnp.ones((16,), jnp.float32)

    @pl.loop(0, N_PAD // 16 + 1)
    def _(i):
        hist_v[pl.ds(i * 16, 16)] = zeros16

    cur[0] = 0
    for g in range(SG):
        g0 = sid * CPS + g * CPG
        pltpu.sync_copy(src_hbm.at[pl.ds(g0, CPG)], sbuf)
        pltpu.sync_copy(dst_hbm.at[pl.ds(g0, CPG)], dbuf)

        @pl.loop(0, CPG)
        def _(j):
            for i in range(CHUNK // 16):
                s16 = sbuf[j, pl.ds(i * 16, 16)]
                d16 = dbuf[j, pl.ds(i * 16, 16)]
                ld16 = d16 - base_row
                m = (ld16 >= 0) & (ld16 < HALF)
                mi = m.astype(jnp.int32)
                pk16 = (s16 << SHIFT) | jnp.where(m, ld16, 0)
                pos16 = cur[0] + jnp.cumsum(mi) - mi
                plsc.store_scatter(plist, [pos16], pk16, mask=m)
                cur[0] = cur[0] + jnp.sum(mi)

        @pl.when(cid == g // (SG // NC))
        def _():
            @pl.loop(0, CPG)
            def _(j):
                for i in range(CHUNK // 16):
                    idx16 = dbuf[j, pl.ds(i * 16, 16)]
                    plsc.addupdate_scatter(hist_v, [idx16], ones16)

    n = cur[0]
    npad = (n + NPM - 1) & (-NPM)
    for g in range(NPM // 16):
        idx16 = n + g * 16 + iota16
        padv = ((N << SHIFT) | (HALF + (g * 16) % CHUNK)) + iota16
        plsc.store_scatter(plist, [idx16], padv, mask=idx16 < npad)

    pltpu.sync_copy(plist, list_out.at[wid])
    cnt_v[pl.ds(0, 16)] = jnp.where(iota16 == 0, npad // CHUNK, 0)
    pltpu.sync_copy(cnt_v, cnt_out.at[wid])
    pltpu.sync_copy(hist_v.at[pl.ds(0, N_PAD)], hist_out.at[wid])


@functools.partial(
    pl.kernel,
    mesh=_mesh,
    out_type=jax.ShapeDtypeStruct((N_PAD, D), jnp.float32),
    compiler_params=_sc_cp_mv,
    scratch_types=[
        pltpu.VMEM((LCAP,), jnp.int32),
        pltpu.VMEM((16,), jnp.int32),
        pltpu.VMEM((NBUF, CHUNK), jnp.int32),
        pltpu.VMEM((NBUF, CHUNK), jnp.int32),
        pltpu.VMEM((NBUF, CHUNK, D), jnp.float32),
        pltpu.VMEM_SHARED((ACC_ROWS, D), jnp.float32),
        pltpu.SemaphoreType.DMA,
        pltpu.SemaphoreType.DMA,
    ],
)
def _sc_matvec(z_hbm, list_hbm, cnt_hbm, out_hbm,
               plist, cnt_v, sidx, didx, rows_v, acc, sem_g, sem_s):
    cid = lax.axis_index("c")
    sid = lax.axis_index("s")
    wid = cid * NS + sid
    base_row = cid * HALF

    pltpu.async_copy(list_hbm.at[wid], plist, sem_g)
    pltpu.sync_copy(cnt_hbm.at[wid], cnt_v)

    zeros16 = jnp.zeros((16,), jnp.float32)

    @pl.loop(0, CHUNK)
    def _(r):
        for i in range(D // 16):
            rows_v[0, r, pl.ds(i * 16, 16)] = zeros16

    l0 = sid * STRIPE
    for k in range(3):
        pltpu.sync_copy(rows_v.at[0], acc.at[pl.ds(l0 + k * CHUNK, CHUNK)])

    nchunks = jnp.sum(cnt_v[pl.ds(0, 16)])
    pltpu.make_async_copy(list_hbm.at[wid], plist, sem_g).wait()
    plsc.subcore_barrier()

    def _unpack(c, b):
        for i in range(CHUNK // 16):
            pk = plist[pl.ds(c * CHUNK + i * 16, 16)]
            sidx[b, pl.ds(i * 16, 16)] = pk >> SHIFT
            didx[b, pl.ds(i * 16, 16)] = pk & DMASK

    def _wait_one(sem):
        pltpu.make_async_copy(z_hbm.at[pl.ds(0, CHUNK)],
                              rows_v.at[0], sem).wait()

    ngroups = nchunks // NBUF

    def _group(gi, carry):
        base = gi * NBUF
        for b in range(NBUF):
            _unpack(base + b, b)
        for b in range(NBUF):
            pltpu.async_copy(z_hbm.at[sidx.at[b]], rows_v.at[b], sem_g)
        for b in range(NBUF):
            _wait_one(sem_g)
            pltpu.async_copy(rows_v.at[b], acc.at[didx.at[b]],
                             sem_s, add=True)
        for b in range(NBUF):
            _wait_one(sem_s)
        return carry

    lax.fori_loop(0, ngroups, _group, 0)

    plsc.subcore_barrier()
    pltpu.sync_copy(acc.at[pl.ds(l0, STRIPE)],
                    out_hbm.at[pl.ds(base_row + l0, STRIPE)])


def _mm1_body(x_ref, w_ref, hist_ref, z_ref, dinv_ref):
    u = jnp.dot(x_ref[...], w_ref[...], preferred_element_type=jnp.float32)
    deg = jnp.sum(hist_ref[...], axis=0) + 1.0
    dinv = lax.rsqrt(deg)[:, None]
    dinv_ref[...] = dinv
    z_ref[...] = u * dinv


_mm1 = pl.pallas_call(
    _mm1_body,
    grid=(GRID,),
    in_specs=[pl.BlockSpec((RB, D), lambda i: (i, 0)),
              pl.BlockSpec((D, D), lambda i: (0, 0)),
              pl.BlockSpec((NW, RB), lambda i: (0, i))],
    out_specs=[pl.BlockSpec((RB, D), lambda i: (i, 0)),
               pl.BlockSpec((RB, 1), lambda i: (i, 0))],
    out_shape=[jax.ShapeDtypeStruct((N_PAD, D), jnp.float32),
               jax.ShapeDtypeStruct((N_PAD, 1), jnp.float32)],
)


def _h_body(t_ref, z1_ref, dinv_ref, b1_ref, z2_ref):
    i = pl.program_id(0)
    dinv = dinv_ref[...]
    pre = dinv * (t_ref[...] + z1_ref[...]) + b1_ref[...]
    h = jnp.where(pre >= 0, pre, 0.01 * pre)
    rows = i * RB + lax.broadcasted_iota(jnp.int32, (RB, 1), 0)
    z2_ref[...] = jnp.where(rows < N, dinv * h, 0.0)


_hstage = pl.pallas_call(
    _h_body,
    grid=(GRID,),
    in_specs=[pl.BlockSpec((RB, D), lambda i: (i, 0)),
              pl.BlockSpec((RB, D), lambda i: (i, 0)),
              pl.BlockSpec((RB, 1), lambda i: (i, 0)),
              pl.BlockSpec((1, D), lambda i: (0, 0))],
    out_specs=pl.BlockSpec((RB, D), lambda i: (i, 0)),
    out_shape=jax.ShapeDtypeStruct((N_PAD, D), jnp.float32),
)


def _out_body(t_ref, z2_ref, dinv_ref, wmu_ref, bmu_ref, wlv_ref, blv_ref,
              omu_ref, olv_ref):
    g = dinv_ref[...] * (t_ref[...] + z2_ref[...])
    omu_ref[...] = jnp.dot(g, wmu_ref[...],
                           preferred_element_type=jnp.float32) + bmu_ref[...]
    olv_ref[...] = jnp.dot(g, wlv_ref[...],
                           preferred_element_type=jnp.float32) + blv_ref[...]


_outstage = pl.pallas_call(
    _out_body,
    grid=(GRID,),
    in_specs=[pl.BlockSpec((RB, D), lambda i: (i, 0)),
              pl.BlockSpec((RB, D), lambda i: (i, 0)),
              pl.BlockSpec((RB, 1), lambda i: (i, 0)),
              pl.BlockSpec((D, OUT), lambda i: (0, 0)),
              pl.BlockSpec((1, OUT), lambda i: (0, 0)),
              pl.BlockSpec((D, OUT), lambda i: (0, 0)),
              pl.BlockSpec((1, OUT), lambda i: (0, 0))],
    out_specs=[pl.BlockSpec((RB, OUT), lambda i: (i, 0)),
               pl.BlockSpec((RB, OUT), lambda i: (i, 0))],
    out_shape=[jax.ShapeDtypeStruct((N, OUT), jnp.float32),
               jax.ShapeDtypeStruct((N, OUT), jnp.float32)],
)


def kernel(x, edge_index, W1, b1, W_mu, b_mu, W_lv, b_lv):
    src = edge_index[0].astype(jnp.int32)
    dst = edge_index[1].astype(jnp.int32)
    src2 = jnp.concatenate(
        [src, jnp.full((E_PAD - E,), N, jnp.int32)]).reshape(NCHUNK, CHUNK)
    dst2 = jnp.concatenate(
        [dst, jnp.full((E_PAD - E,), N_PAD, jnp.int32)]).reshape(NCHUNK, CHUNK)
    x_pad = jnp.concatenate(
        [x, jnp.zeros((N_PAD - N, D), jnp.float32)], axis=0)

    hist, plists, cnts = _sc_prep(src2, dst2)
    z1, dinv = _mm1(x_pad, W1, hist)
    t1 = _sc_matvec(z1, plists, cnts)
    z2 = _hstage(t1, z1, dinv, b1.reshape(1, D))
    t2 = _sc_matvec(z2, plists, cnts)
    mu, lv = _outstage(t2, z2, dinv, W_mu, b_mu.reshape(1, OUT),
                       W_lv, b_lv.reshape(1, OUT))
    return mu, lv


# speedup vs baseline: 1.9371x; 1.9371x over previous
import dataclasses
import functools

import jax
import jax.numpy as jnp
from jax import lax
from jax.experimental import pallas as pl
from jax.experimental.pallas import tpu as pltpu
from jax.experimental.pallas import tpu_sc as plsc

N = 10000
D = 128
OUT = 64
E = 320000
NC, NS = 2, 16
NW = NC * NS
CHUNK = 128
E_PAD = 327680
NCHUNK = E_PAD // CHUNK
CPW_H = NCHUNK // NW
CPS = NCHUNK // NS
EPS = CPS * CHUNK
NBUF = 4
N_PAD = 10240
HALF = N_PAD // NC
ACC_ROWS = HALF + CHUNK
STRIPE = HALF // NS
LCAP = EPS + CHUNK
SG = 4
CPG = CPS // SG
SHIFT = 14
DMASK = (1 << SHIFT) - 1
RB = 2048
GRID = N_PAD // RB

_mesh = plsc.VectorSubcoreMesh(core_axis_name="c", subcore_axis_name="s")

_sc_cp = pltpu.CompilerParams()
if "needs_layout_passes" in pltpu.CompilerParams.__dataclass_fields__:
    _sc_cp = dataclasses.replace(_sc_cp, needs_layout_passes=False)
_sc_cp_mv = dataclasses.replace(_sc_cp, use_tc_tiling_on_sc=False)


@functools.partial(
    pl.kernel,
    mesh=_mesh,
    out_type=(
        jax.ShapeDtypeStruct((NW, N_PAD), jnp.float32),
        jax.ShapeDtypeStruct((NW, LCAP), jnp.int32),
        jax.ShapeDtypeStruct((NW, 16), jnp.int32),
    ),
    compiler_params=_sc_cp,
    scratch_types=[
        pltpu.VMEM((CPG, CHUNK), jnp.int32),
        pltpu.VMEM((CPG, CHUNK), jnp.int32),
        pltpu.VMEM((LCAP,), jnp.int32),
        pltpu.VMEM((N_PAD + 16,), jnp.float32),
        pltpu.VMEM((16,), jnp.int32),
        pltpu.SMEM((1,), jnp.int32),
    ],
)
def _sc_prep(src_hbm, dst_hbm, hist_out, list_out, cnt_out,
             sbuf, dbuf, plist, hist_v, cnt_v, cur):
    cid = lax.axis_index("c")
    sid = lax.axis_index("s")
    wid = cid * NS + sid
    base_row = cid * HALF
    iota16 = lax.iota(jnp.int32, 16)
    zeros16 = jnp.zeros((16,), jnp.float32)
    ones16 = jnp.ones((16,), jnp.float32)

    @pl.loop(0, N_PAD // 16 + 1)
    def _(i):
        hist_v[pl.ds(i * 16, 16)] = zeros16

    cur[0] = 0
    for g in range(SG):
        g0 = sid * CPS + g * CPG
        pltpu.sync_copy(src_hbm.at[pl.ds(g0, CPG)], sbuf)
        pltpu.sync_copy(dst_hbm.at[pl.ds(g0, CPG)], dbuf)

        @pl.loop(0, CPG)
        def _(j):
            for i in range(CHUNK // 16):
                s16 = sbuf[j, pl.ds(i * 16, 16)]
                d16 = dbuf[j, pl.ds(i * 16, 16)]
                ld16 = d16 - base_row
                m = (ld16 >= 0) & (ld16 < HALF)
                mi = m.astype(jnp.int32)
                pk16 = (s16 << SHIFT) | jnp.where(m, ld16, 0)
                pos16 = cur[0] + jnp.cumsum(mi) - mi
                plsc.store_scatter(plist, [pos16], pk16, mask=m)
                cur[0] = cur[0] + jnp.sum(mi)

        @pl.when(cid == g // (SG // NC))
        def _():
            @pl.loop(0, CPG)
            def _(j):
                for i in range(CHUNK // 16):
                    idx16 = dbuf[j, pl.ds(i * 16, 16)]
                    plsc.addupdate_scatter(hist_v, [idx16], ones16)

    n = cur[0]
    npad = (n + CHUNK - 1) & (-CHUNK)
    for g in range(CHUNK // 16):
        idx16 = n + g * 16 + iota16
        padv = ((N << SHIFT) | (HALF + g * 16)) + iota16
        plsc.store_scatter(plist, [idx16], padv, mask=idx16 < npad)

    pltpu.sync_copy(plist, list_out.at[wid])
    cnt_v[pl.ds(0, 16)] = jnp.where(iota16 == 0, npad // CHUNK, 0)
    pltpu.sync_copy(cnt_v, cnt_out.at[wid])
    pltpu.sync_copy(hist_v.at[pl.ds(0, N_PAD)], hist_out.at[wid])


@functools.partial(
    pl.kernel,
    mesh=_mesh,
    out_type=jax.ShapeDtypeStruct((N_PAD, D), jnp.float32),
    compiler_params=_sc_cp_mv,
    scratch_types=[
        pltpu.VMEM((LCAP,), jnp.int32),
        pltpu.VMEM((16,), jnp.int32),
        pltpu.VMEM((NBUF, CHUNK), jnp.int32),
        pltpu.VMEM((NBUF, CHUNK), jnp.int32),
        pltpu.VMEM((NBUF, CHUNK, D), jnp.float32),
        pltpu.VMEM_SHARED((ACC_ROWS, D), jnp.float32),
        pltpu.SemaphoreType.DMA,
        pltpu.SemaphoreType.DMA,
    ],
)
def _sc_matvec(z_hbm, list_hbm, cnt_hbm, out_hbm,
               plist, cnt_v, sidx, didx, rows_v, acc, sem_g, sem_s):
    cid = lax.axis_index("c")
    sid = lax.axis_index("s")
    wid = cid * NS + sid
    base_row = cid * HALF

    pltpu.async_copy(list_hbm.at[wid], plist, sem_g)
    pltpu.sync_copy(cnt_hbm.at[wid], cnt_v)

    zeros16 = jnp.zeros((16,), jnp.float32)

    @pl.loop(0, CHUNK)
    def _(r):
        for i in range(D // 16):
            rows_v[0, r, pl.ds(i * 16, 16)] = zeros16

    l0 = sid * STRIPE
    for k in range(3):
        pltpu.sync_copy(rows_v.at[0], acc.at[pl.ds(l0 + k * CHUNK, CHUNK)])

    nchunks = jnp.sum(cnt_v[pl.ds(0, 16)])
    pltpu.make_async_copy(list_hbm.at[wid], plist, sem_g).wait()
    plsc.subcore_barrier()

    def _unpack(c, b):
        for i in range(CHUNK // 16):
            pk = plist[pl.ds(c * CHUNK + i * 16, 16)]
            sidx[b, pl.ds(i * 16, 16)] = pk >> SHIFT
            didx[b, pl.ds(i * 16, 16)] = pk & DMASK

    def _wait_one(sem):
        pltpu.make_async_copy(z_hbm.at[pl.ds(0, CHUNK)],
                              rows_v.at[0], sem).wait()

    ngroups = nchunks // NBUF
    rem = nchunks - ngroups * NBUF

    def _group(gi, carry):
        base = gi * NBUF
        for b in range(NBUF):
            _unpack(base + b, b)
        for b in range(NBUF):
            pltpu.async_copy(z_hbm.at[sidx.at[b]], rows_v.at[b], sem_g)
        for b in range(NBUF):
            _wait_one(sem_g)
            pltpu.async_copy(rows_v.at[b], acc.at[didx.at[b]],
                             sem_s, add=True)
        for b in range(NBUF):
            _wait_one(sem_s)
        return carry

    lax.fori_loop(0, ngroups, _group, 0)

    for b in range(NBUF):
        @pl.when(b < rem)
        def _():
            c = ngroups * NBUF + b
            _unpack(c, b)
            pltpu.sync_copy(z_hbm.at[sidx.at[b]], rows_v.at[b])
            pltpu.sync_copy(rows_v.at[b], acc.at[didx.at[b]], add=True)

    plsc.subcore_barrier()
    pltpu.sync_copy(acc.at[pl.ds(l0, STRIPE)],
                    out_hbm.at[pl.ds(base_row + l0, STRIPE)])


def _mm1_body(x_ref, w_ref, hist_ref, z_ref, dinv_ref):
    u = jnp.dot(x_ref[...], w_ref[...], preferred_element_type=jnp.float32)
    deg = jnp.sum(hist_ref[...], axis=0) + 1.0
    dinv = lax.rsqrt(deg)[:, None]
    dinv_ref[...] = dinv
    z_ref[...] = u * dinv


_mm1 = pl.pallas_call(
    _mm1_body,
    grid=(GRID,),
    in_specs=[pl.BlockSpec((RB, D), lambda i: (i, 0)),
              pl.BlockSpec((D, D), lambda i: (0, 0)),
              pl.BlockSpec((NW, RB), lambda i: (0, i))],
    out_specs=[pl.BlockSpec((RB, D), lambda i: (i, 0)),
               pl.BlockSpec((RB, 1), lambda i: (i, 0))],
    out_shape=[jax.ShapeDtypeStruct((N_PAD, D), jnp.float32),
               jax.ShapeDtypeStruct((N_PAD, 1), jnp.float32)],
)


def _h_body(t_ref, z1_ref, dinv_ref, b1_ref, z2_ref):
    i = pl.program_id(0)
    dinv = dinv_ref[...]
    pre = dinv * (t_ref[...] + z1_ref[...]) + b1_ref[...]
    h = jnp.where(pre >= 0, pre, 0.01 * pre)
    rows = i * RB + lax.broadcasted_iota(jnp.int32, (RB, 1), 0)
    z2_ref[...] = jnp.where(rows < N, dinv * h, 0.0)


_hstage = pl.pallas_call(
    _h_body,
    grid=(GRID,),
    in_specs=[pl.BlockSpec((RB, D), lambda i: (i, 0)),
              pl.BlockSpec((RB, D), lambda i: (i, 0)),
              pl.BlockSpec((RB, 1), lambda i: (i, 0)),
              pl.BlockSpec((1, D), lambda i: (0, 0))],
    out_specs=pl.BlockSpec((RB, D), lambda i: (i, 0)),
    out_shape=jax.ShapeDtypeStruct((N_PAD, D), jnp.float32),
)


def _out_body(t_ref, z2_ref, dinv_ref, wmu_ref, bmu_ref, wlv_ref, blv_ref,
              omu_ref, olv_ref):
    g = dinv_ref[...] * (t_ref[...] + z2_ref[...])
    omu_ref[...] = jnp.dot(g, wmu_ref[...],
                           preferred_element_type=jnp.float32) + bmu_ref[...]
    olv_ref[...] = jnp.dot(g, wlv_ref[...],
                           preferred_element_type=jnp.float32) + blv_ref[...]


_outstage = pl.pallas_call(
    _out_body,
    grid=(GRID,),
    in_specs=[pl.BlockSpec((RB, D), lambda i: (i, 0)),
              pl.BlockSpec((RB, D), lambda i: (i, 0)),
              pl.BlockSpec((RB, 1), lambda i: (i, 0)),
              pl.BlockSpec((D, OUT), lambda i: (0, 0)),
              pl.BlockSpec((1, OUT), lambda i: (0, 0)),
              pl.BlockSpec((D, OUT), lambda i: (0, 0)),
              pl.BlockSpec((1, OUT), lambda i: (0, 0))],
    out_specs=[pl.BlockSpec((RB, OUT), lambda i: (i, 0)),
               pl.BlockSpec((RB, OUT), lambda i: (i, 0))],
    out_shape=[jax.ShapeDtypeStruct((N, OUT), jnp.float32),
               jax.ShapeDtypeStruct((N, OUT), jnp.float32)],
)


def kernel(x, edge_index, W1, b1, W_mu, b_mu, W_lv, b_lv):
    src = edge_index[0].astype(jnp.int32)
    dst = edge_index[1].astype(jnp.int32)
    src2 = jnp.concatenate(
        [src, jnp.full((E_PAD - E,), N, jnp.int32)]).reshape(NCHUNK, CHUNK)
    dst2 = jnp.concatenate(
        [dst, jnp.full((E_PAD - E,), N_PAD, jnp.int32)]).reshape(NCHUNK, CHUNK)
    x_pad = jnp.concatenate(
        [x, jnp.zeros((N_PAD - N, D), jnp.float32)], axis=0)

    hist, plists, cnts = _sc_prep(src2, dst2)
    z1, dinv = _mm1(x_pad, W1, hist)
    t1 = _sc_matvec(z1, plists, cnts)
    z2 = _hstage(t1, z1, dinv, b1.reshape(1, D))
    t2 = _sc_matvec(z2, plists, cnts)
    mu, lv = _outstage(t2, z2, dinv, W_mu, b_mu.reshape(1, OUT),
                       W_lv, b_lv.reshape(1, OUT))
    return mu, lv
